# baseline (speedup 1.0000x reference)
import functools

import jax
import jax.numpy as jnp
from jax import lax
from jax.experimental import pallas as pl
from jax.experimental.pallas import tpu as pltpu
from jax.experimental.pallas import tpu_sc as plsc

N = 10000
C = 64
E = 160000
NITER = 10
NCORES = 2
NSUB = 16
NTILES = NCORES * NSUB
CH = 80
NCHUNK = 64
NB = 2
EPT = CH * NCHUNK
EPAD = EPT * NTILES
RPT = 632
NP = RPT * NSUB
PAD_ROW = N
CB = [(0, 80), (80, 80), (160, 80), (240, 80),
      (320, 80), (400, 80), (480, 80), (560, 72)]

ROWB = 632
FROWB = 400

_sc_mesh = plsc.VectorSubcoreMesh(core_axis_name="c", subcore_axis_name="s")

_sc_scratch = [
    pltpu.VMEM((NCHUNK, CH), jnp.int32),
    pltpu.VMEM((NCHUNK, CH), jnp.int32),
    pltpu.VMEM((CH, C), jnp.float32),
    pltpu.VMEM((CH, C), jnp.float32),
    pltpu.VMEM((80, NCORES, C), jnp.float32),
    pltpu.VMEM((80, NCORES, C), jnp.float32),
    pltpu.VMEM((80, C), jnp.float32),
    pltpu.VMEM_SHARED((NP, C), jnp.float32),
    pltpu.VMEM_SHARED((NP, C), jnp.float32),
    pltpu.SemaphoreType.DMA,
    pltpu.SemaphoreType.DMA,
    pltpu.SemaphoreType.DMA,
    pltpu.SemaphoreType.DMA,
    pltpu.SemaphoreType.REGULAR,
]


def _aggregate_phase(dstv, srcv, gb, gsem, ssem, ysh, acc):
    for j in range(NB - 1):
        pltpu.async_copy(ysh.at[dstv.at[j]], gb[j], gsem[j])
    for j in range(NCHUNK):
        b = j % NB
        pltpu.make_async_copy(ysh.at[dstv.at[j]], gb[b], gsem[b]).wait()
        nj = j + NB - 1
        if nj < NCHUNK:
            bn = nj % NB
            if nj - NB >= 0:
                pltpu.make_async_copy(
                    gb[bn], acc.at[srcv.at[nj - NB]], ssem[bn]).wait()
            pltpu.async_copy(ysh.at[dstv.at[nj]], gb[bn], gsem[bn])
        pltpu.async_copy(gb[b], acc.at[srcv.at[j]], ssem[b], add=True)
    for c in range(max(0, NCHUNK - NB), NCHUNK):
        b = c % NB
        pltpu.make_async_copy(gb[b], acc.at[srcv.at[c]], ssem[b]).wait()


@functools.partial(
    pl.kernel,
    out_type=jax.ShapeDtypeStruct((NCORES, NP, C), jnp.float32),
    mesh=_sc_mesh,
    scratch_types=_sc_scratch,
    compiler_params=pltpu.CompilerParams(use_tc_tiling_on_sc=False),
)
def _sc_aggregate(y_hbm, dst_hbm, src_hbm, zeros_hbm, out_hbm,
                  dstv, srcv, gb0, gb1, pxb, cmb, yb, acc, ysh, *sems):
    gsem, ssem = sems[:NB], sems[NB:2 * NB]
    cid = lax.axis_index("c")
    sid = lax.axis_index("s")
    wid = cid * NSUB + sid
    pltpu.sync_copy(dst_hbm.at[wid], dstv)
    pltpu.sync_copy(src_hbm.at[wid], srcv)
    pltpu.sync_copy(zeros_hbm, acc.at[pl.ds(sid * RPT, RPT)])
    pltpu.sync_copy(y_hbm.at[pl.ds(sid * RPT, RPT)],
                    ysh.at[pl.ds(sid * RPT, RPT)])
    plsc.subcore_barrier()
    _aggregate_phase(dstv, srcv, (gb0, gb1), gsem, ssem, ysh, acc)
    plsc.subcore_barrier()
    pltpu.sync_copy(acc.at[pl.ds(sid * RPT, RPT)],
                    out_hbm.at[cid, pl.ds(sid * RPT, RPT)])


@functools.partial(
    pl.kernel,
    out_type=(jax.ShapeDtypeStruct((NP, C), jnp.float32),
              jax.ShapeDtypeStruct((NP, NCORES, C), jnp.float32)),
    mesh=_sc_mesh,
    scratch_types=_sc_scratch,
    compiler_params=pltpu.CompilerParams(use_tc_tiling_on_sc=False),
)
def _sc_ppr(y0_hbm, cm_hbm, dst_hbm, src_hbm, zeros_hbm,
            yout_hbm, pex_hbm,
            dstv, srcv, gb0, gb1, pxb, cmb, yb, acc, ysh, *sems):
    gsem, ssem = sems[:NB], sems[NB:2 * NB]
    xsem = sems[2 * NB]
    cid = lax.axis_index("c")
    sid = lax.axis_index("s")
    wid = cid * NSUB + sid
    r0 = sid * RPT
    pltpu.sync_copy(dst_hbm.at[wid], dstv)
    pltpu.sync_copy(src_hbm.at[wid], srcv)
    pltpu.sync_copy(y0_hbm.at[pl.ds(r0, RPT)], ysh.at[pl.ds(r0, RPT)])

    def _iter(k, carry):
        pltpu.sync_copy(zeros_hbm, acc.at[pl.ds(sid * RPT, RPT)])
        plsc.subcore_barrier()
        _aggregate_phase(dstv, srcv, (gb0, gb1), gsem, ssem, ysh, acc)
        plsc.subcore_barrier()
        pltpu.sync_copy(acc.at[pl.ds(sid * RPT, RPT)],
                        pex_hbm.at[pl.ds(sid * RPT, RPT), cid])
        plsc.subcore_barrier()
        pl.semaphore_signal(xsem, 1, core_index=1 - cid)
        pl.semaphore_wait(xsem, 1)
        plsc.subcore_barrier()
        for (off, sz) in CB:
            pltpu.sync_copy(pex_hbm.at[pl.ds(r0 + off, sz)],
                            pxb.at[pl.ds(0, sz)])
            pltpu.sync_copy(cm_hbm.at[pl.ds(r0 + off, sz)],
                            cmb.at[pl.ds(0, sz)])
            pltpu.sync_copy(ysh.at[pl.ds(r0 + off, sz)], yb.at[pl.ds(0, sz)])

            def _row(r, rc):
                for h in range(C // 16):
                    s = pl.ds(h * 16, 16)
                    t = pxb[r, 0, s] + pxb[r, 1, s] + yb[r, s]
                    yb[r, s] = cmb[r, 0, s] * t + cmb[r, 1, s]
                return rc

            lax.fori_loop(0, sz, _row, 0)
            pltpu.sync_copy(yb.at[pl.ds(0, sz)], ysh.at[pl.ds(r0 + off, sz)])

            @pl.when(jnp.logical_and(k == NITER - 1, cid == 0))
            def _():
                pltpu.sync_copy(yb.at[pl.ds(0, sz)],
                                yout_hbm.at[pl.ds(r0 + off, sz)])

        return carry

    lax.fori_loop(0, NITER, _iter, 0)


def _dot(a, b):
    return jnp.dot(a, b, preferred_element_type=jnp.float32,
                   precision=lax.Precision.HIGHEST)


def _mlp_body(attr_ref, w0_ref, w1_ref, w2_ref, l_ref):
    x = jnp.maximum(_dot(attr_ref[...], w0_ref[...]), 0.0)
    h = jnp.maximum(_dot(x, w1_ref[...]), 0.0)
    l_ref[...] = _dot(h, w2_ref[...])


def _prep_body(l_ref, pdeg_ref, y0_ref, cm_ref, sq_ref):
    deg = pdeg_ref[0] + pdeg_ref[1] + 1.0
    dinv = lax.rsqrt(deg)
    sq_ref[...] = deg * dinv
    y0 = dinv * l_ref[...]
    y0_ref[...] = y0
    cm_ref[...] = jnp.stack([0.9 / deg, 0.1 * y0], axis=1)


def _final_body(y_ref, sq_ref, o_ref):
    preds = sq_ref[...] * y_ref[...]
    sh = preds - jnp.max(preds, axis=1, keepdims=True)
    o_ref[...] = sh - jnp.log(jnp.sum(jnp.exp(sh), axis=1, keepdims=True))


_mblk = lambda: pl.BlockSpec((ROWB, C), lambda i: (i, 0))

_mlp = pl.pallas_call(
    _mlp_body,
    grid=(NP // ROWB,),
    in_specs=[
        pl.BlockSpec((ROWB, 256), lambda i: (i, 0)),
        pl.BlockSpec((256, 512), lambda i: (0, 0)),
        pl.BlockSpec((512, 256), lambda i: (0, 0)),
        pl.BlockSpec((256, C), lambda i: (0, 0)),
    ],
    out_specs=_mblk(),
    out_shape=jax.ShapeDtypeStruct((NP, C), jnp.float32),
)

_prep = pl.pallas_call(
    _prep_body,
    grid=(NP // ROWB,),
    in_specs=[_mblk(), pl.BlockSpec((NCORES, ROWB, C), lambda i: (0, i, 0))],
    out_specs=[_mblk(), pl.BlockSpec((ROWB, NCORES, C), lambda i: (i, 0, 0)),
               _mblk()],
    out_shape=[jax.ShapeDtypeStruct((NP, C), jnp.float32),
               jax.ShapeDtypeStruct((NP, NCORES, C), jnp.float32),
               jax.ShapeDtypeStruct((NP, C), jnp.float32)],
)

_final = pl.pallas_call(
    _final_body,
    grid=(N // FROWB,),
    in_specs=[pl.BlockSpec((FROWB, C), lambda i: (i, 0)),
              pl.BlockSpec((FROWB, C), lambda i: (i, 0))],
    out_specs=pl.BlockSpec((FROWB, C), lambda i: (i, 0)),
    out_shape=jax.ShapeDtypeStruct((N, C), jnp.float32),
)


def kernel(adj_dense, attr_matrix, test, epochs, edge_index, W0, W1, W2):
    src = edge_index[0].astype(jnp.int32)
    dst = edge_index[1].astype(jnp.int32)
    npad = EPAD - E
    src_t = jnp.concatenate(
        [src, jnp.full((npad,), PAD_ROW, jnp.int32)]).reshape(NTILES, NCHUNK, CH)
    dst_t = jnp.concatenate(
        [dst, jnp.zeros((npad,), jnp.int32)]).reshape(NTILES, NCHUNK, CH)
    zeros_tile = jnp.zeros((RPT, C), jnp.float32)
    ones_y = jnp.ones((NP, C), jnp.float32)

    pdeg = _sc_aggregate(ones_y, dst_t, src_t, zeros_tile)
    logits = _mlp(attr_matrix, W0, W1, W2)
    y0, cm, sq_w = _prep(logits, pdeg)
    y10, _ = _sc_ppr(y0, cm, dst_t, src_t, zeros_tile)
    return _final(y10, sq_w)


# speedup vs baseline: 1.2153x; 1.2153x over previous
import functools

import jax
import jax.numpy as jnp
from jax import lax
from jax.experimental import pallas as pl
from jax.experimental.pallas import tpu as pltpu
from jax.experimental.pallas import tpu_sc as plsc

N = 10000
C = 64
E = 160000
NCORES = 2
NSUB = 16
NTILES = NCORES * NSUB
CH = 160
NCHUNK = 32
NB = 2
EPT = CH * NCHUNK
EPAD = EPT * NTILES
RPT = 632
NP = RPT * NSUB
PAD_ROW = N
CB = [(0, 88), (88, 88), (176, 88), (264, 88),
      (352, 88), (440, 88), (528, 88), (616, 16)]

ROWB = 632
FROWB = 400

_sc_mesh = plsc.VectorSubcoreMesh(core_axis_name="c", subcore_axis_name="s")

_sc_scratch = [
    pltpu.VMEM((NCHUNK, CH), jnp.int32),
    pltpu.VMEM((NCHUNK, CH), jnp.int32),
    pltpu.VMEM((CH, C), jnp.float32),
    pltpu.VMEM((CH, C), jnp.float32),
    pltpu.VMEM((88, C), jnp.float32),
    pltpu.VMEM((88, C), jnp.float32),
    pltpu.VMEM((88, C), jnp.float32),
    pltpu.VMEM_SHARED((NP, C), jnp.float32),
    pltpu.VMEM_SHARED((NP, C), jnp.float32),
    pltpu.SemaphoreType.DMA,
    pltpu.SemaphoreType.DMA,
    pltpu.SemaphoreType.DMA,
    pltpu.SemaphoreType.DMA,
]


def _aggregate_phase(dstv, srcv, gb, gsem, ssem, ysh, acc, out_hbm, cid, sid):
    for j in range(NB - 1):
        pltpu.async_copy(ysh.at[dstv.at[j]], gb[j], gsem[j])
    for j in range(NCHUNK):
        b = j % NB
        pltpu.make_async_copy(ysh.at[dstv.at[j]], gb[b], gsem[b]).wait()
        nj = j + NB - 1
        if nj < NCHUNK:
            bn = nj % NB
            if nj - NB >= 0:
                pltpu.make_async_copy(
                    gb[bn], acc.at[srcv.at[nj - NB]], ssem[bn]).wait()
            pltpu.async_copy(ysh.at[dstv.at[nj]], gb[bn], gsem[bn])
        pltpu.async_copy(gb[b], acc.at[srcv.at[j]], ssem[b], add=True)
    for c in range(max(0, NCHUNK - NB), NCHUNK):
        b = c % NB
        pltpu.make_async_copy(gb[b], acc.at[srcv.at[c]], ssem[b]).wait()
    plsc.subcore_barrier()
    pltpu.sync_copy(acc.at[pl.ds(sid * RPT, RPT)],
                    out_hbm.at[cid, pl.ds(sid * RPT, RPT)])


def _load_edges_and_zero(dst_hbm, src_hbm, zeros_hbm, dstv, srcv, acc, sid,
                         wid):
    pltpu.sync_copy(dst_hbm.at[wid], dstv)
    pltpu.sync_copy(src_hbm.at[wid], srcv)
    pltpu.sync_copy(zeros_hbm, acc.at[pl.ds(sid * RPT, RPT)])


@functools.partial(
    pl.kernel,
    out_type=jax.ShapeDtypeStruct((NCORES, NP, C), jnp.float32),
    mesh=_sc_mesh,
    scratch_types=_sc_scratch,
    compiler_params=pltpu.CompilerParams(use_tc_tiling_on_sc=False),
)
def _sc_aggregate(y_hbm, dst_hbm, src_hbm, zeros_hbm, out_hbm,
                  dstv, srcv, gb0, gb1, yb, cb, mb, acc, ysh, *sems):
    gsem, ssem = sems[:NB], sems[NB:]
    cid = lax.axis_index("c")
    sid = lax.axis_index("s")
    wid = cid * NSUB + sid
    _load_edges_and_zero(dst_hbm, src_hbm, zeros_hbm, dstv, srcv, acc, sid,
                         wid)
    pltpu.sync_copy(y_hbm.at[pl.ds(sid * RPT, RPT)],
                    ysh.at[pl.ds(sid * RPT, RPT)])
    plsc.subcore_barrier()
    _aggregate_phase(dstv, srcv, (gb0, gb1), gsem, ssem, ysh, acc, out_hbm,
                     cid, sid)


def _dot(a, b):
    return jnp.dot(a, b, preferred_element_type=jnp.float32,
                   precision=lax.Precision.HIGHEST)


def _mlp_body(attr_ref, w0_ref, w1_ref, w2_ref, l_ref):
    x = jnp.maximum(_dot(attr_ref[...], w0_ref[...]), 0.0)
    h = jnp.maximum(_dot(x, w1_ref[...]), 0.0)
    l_ref[...] = _dot(h, w2_ref[...])


def _prep_body(l_ref, pdeg_ref, y0_ref, m_ref, c_ref, sq_ref):
    deg = pdeg_ref[0] + pdeg_ref[1] + 1.0
    dinv = lax.rsqrt(deg)
    c_ref[...] = 0.9 / deg
    sq_ref[...] = deg * dinv
    y0 = dinv * l_ref[...]
    y0_ref[...] = y0
    m_ref[...] = 0.1 * y0


def _final_body(p_ref, y_ref, c_ref, m_ref, sq_ref, o_ref):
    t = c_ref[...] * (p_ref[0] + p_ref[1] + y_ref[...]) + m_ref[...]
    preds = sq_ref[...] * t
    sh = preds - jnp.max(preds, axis=1, keepdims=True)
    o_ref[...] = sh - jnp.log(jnp.sum(jnp.exp(sh), axis=1, keepdims=True))


_mblk = lambda: pl.BlockSpec((ROWB, C), lambda i: (i, 0))

_mlp = pl.pallas_call(
    _mlp_body,
    grid=(NP // ROWB,),
    in_specs=[
        pl.BlockSpec((ROWB, 256), lambda i: (i, 0)),
        pl.BlockSpec((256, 512), lambda i: (0, 0)),
        pl.BlockSpec((512, 256), lambda i: (0, 0)),
        pl.BlockSpec((256, C), lambda i: (0, 0)),
    ],
    out_specs=_mblk(),
    out_shape=jax.ShapeDtypeStruct((NP, C), jnp.float32),
)

_prep = pl.pallas_call(
    _prep_body,
    grid=(NP // ROWB,),
    in_specs=[_mblk(), pl.BlockSpec((NCORES, ROWB, C), lambda i: (0, i, 0))],
    out_specs=[_mblk(), _mblk(), _mblk(), _mblk()],
    out_shape=[jax.ShapeDtypeStruct((NP, C), jnp.float32)] * 4,
)

_fblk = lambda: pl.BlockSpec((FROWB, C), lambda i: (i, 0))

_final = pl.pallas_call(
    _final_body,
    grid=(N // FROWB,),
    in_specs=[pl.BlockSpec((NCORES, FROWB, C), lambda i: (0, i, 0)),
              _fblk(), _fblk(), _fblk(), _fblk()],
    out_specs=_fblk(),
    out_shape=jax.ShapeDtypeStruct((N, C), jnp.float32),
)


def kernel(adj_dense, attr_matrix, test, epochs, edge_index, W0, W1, W2):
    src = edge_index[0].astype(jnp.int32)
    dst = edge_index[1].astype(jnp.int32)
    npad = EPAD - E
    src_t = jnp.concatenate(
        [src, jnp.full((npad,), PAD_ROW, jnp.int32)]).reshape(NTILES, NCHUNK, CH)
    dst_t = jnp.concatenate(
        [dst, jnp.zeros((npad,), jnp.int32)]).reshape(NTILES, NCHUNK, CH)
    zeros_tile = jnp.zeros((RPT, C), jnp.float32)
    ones_y = jnp.ones((NP, C), jnp.float32)

    pdeg = _sc_aggregate(ones_y, dst_t, src_t, zeros_tile)
    logits = _mlp(attr_matrix, W0, W1, W2)
    y, m, c_w, sq_w = _prep(logits, pdeg)
    for k in range(10):
        p = _sc_aggregate(y, dst_t, src_t, zeros_tile)
        if k < 9:
            y = c_w * (p[0] + p[1] + y) + m
        else:
            out = _final(p, y, c_w, m, sq_w)
    return out


# speedup vs baseline: 1.2289x; 1.0113x over previous
import functools

import jax
import jax.numpy as jnp
from jax import lax
from jax.experimental import pallas as pl
from jax.experimental.pallas import tpu as pltpu
from jax.experimental.pallas import tpu_sc as plsc

N = 10000
C = 64
E = 160000
NCORES = 2
NSUB = 16
NTILES = NCORES * NSUB
CH = 160
NCHUNK = 32
NB = 3
EPT = CH * NCHUNK
EPAD = EPT * NTILES
RPT = 632
NP = RPT * NSUB
PAD_ROW = N
CB = [(0, 88), (88, 88), (176, 88), (264, 88),
      (352, 88), (440, 88), (528, 88), (616, 16)]

ROWB = 632
FROWB = 400

_sc_mesh = plsc.VectorSubcoreMesh(core_axis_name="c", subcore_axis_name="s")

_sc_scratch = [
    pltpu.VMEM((NCHUNK, CH), jnp.int32),
    pltpu.VMEM((NCHUNK, CH), jnp.int32),
    pltpu.VMEM((CH, C), jnp.float32),
    pltpu.VMEM((CH, C), jnp.float32),
    pltpu.VMEM((CH, C), jnp.float32),
    pltpu.VMEM_SHARED((NP, C), jnp.float32),
    pltpu.VMEM_SHARED((NP, C), jnp.float32),
] + [pltpu.SemaphoreType.DMA] * 6


def _aggregate_phase(dstv, srcv, gb, gsem, ssem, ysh, acc, out_hbm, cid, sid):
    for j in range(NB - 1):
        pltpu.async_copy(ysh.at[dstv.at[j]], gb[j], gsem[j])
    for j in range(NCHUNK):
        b = j % NB
        pltpu.make_async_copy(ysh.at[dstv.at[j]], gb[b], gsem[b]).wait()
        nj = j + NB - 1
        if nj < NCHUNK:
            bn = nj % NB
            if nj - NB >= 0:
                pltpu.make_async_copy(
                    gb[bn], acc.at[srcv.at[nj - NB]], ssem[bn]).wait()
            pltpu.async_copy(ysh.at[dstv.at[nj]], gb[bn], gsem[bn])
        pltpu.async_copy(gb[b], acc.at[srcv.at[j]], ssem[b], add=True)
    for c in range(max(0, NCHUNK - NB), NCHUNK):
        b = c % NB
        pltpu.make_async_copy(gb[b], acc.at[srcv.at[c]], ssem[b]).wait()
    plsc.subcore_barrier()
    pltpu.sync_copy(acc.at[pl.ds(sid * RPT, RPT)],
                    out_hbm.at[cid, pl.ds(sid * RPT, RPT)])


def _load_edges_and_zero(dst_hbm, src_hbm, zeros_hbm, dstv, srcv, acc, sid,
                         wid):
    pltpu.sync_copy(dst_hbm.at[wid], dstv)
    pltpu.sync_copy(src_hbm.at[wid], srcv)
    pltpu.sync_copy(zeros_hbm, acc.at[pl.ds(sid * RPT, RPT)])


@functools.partial(
    pl.kernel,
    out_type=jax.ShapeDtypeStruct((NCORES, NP, C), jnp.float32),
    mesh=_sc_mesh,
    scratch_types=_sc_scratch,
    compiler_params=pltpu.CompilerParams(use_tc_tiling_on_sc=False),
)
def _sc_aggregate(y_hbm, dst_hbm, src_hbm, zeros_hbm, out_hbm,
                  dstv, srcv, gb0, gb1, gb2, acc, ysh, *sems):
    gsem, ssem = sems[:NB], sems[NB:]
    cid = lax.axis_index("c")
    sid = lax.axis_index("s")
    wid = cid * NSUB + sid
    _load_edges_and_zero(dst_hbm, src_hbm, zeros_hbm, dstv, srcv, acc, sid,
                         wid)
    pltpu.sync_copy(y_hbm.at[pl.ds(sid * RPT, RPT)],
                    ysh.at[pl.ds(sid * RPT, RPT)])
    plsc.subcore_barrier()
    _aggregate_phase(dstv, srcv, (gb0, gb1, gb2), gsem, ssem, ysh, acc, out_hbm,
                     cid, sid)


def _dot(a, b):
    return jnp.dot(a, b, preferred_element_type=jnp.float32,
                   precision=lax.Precision.HIGHEST)


def _mlp_body(attr_ref, w0_ref, w1_ref, w2_ref, l_ref):
    x = jnp.maximum(_dot(attr_ref[...], w0_ref[...]), 0.0)
    h = jnp.maximum(_dot(x, w1_ref[...]), 0.0)
    l_ref[...] = _dot(h, w2_ref[...])


def _prep_body(l_ref, pdeg_ref, y0_ref, m_ref, c_ref, sq_ref):
    deg = pdeg_ref[0] + pdeg_ref[1] + 1.0
    dinv = lax.rsqrt(deg)
    c_ref[...] = 0.9 / deg
    sq_ref[...] = deg * dinv
    y0 = dinv * l_ref[...]
    y0_ref[...] = y0
    m_ref[...] = 0.1 * y0


def _final_body(p_ref, y_ref, c_ref, m_ref, sq_ref, o_ref):
    t = c_ref[...] * (p_ref[0] + p_ref[1] + y_ref[...]) + m_ref[...]
    preds = sq_ref[...] * t
    sh = preds - jnp.max(preds, axis=1, keepdims=True)
    o_ref[...] = sh - jnp.log(jnp.sum(jnp.exp(sh), axis=1, keepdims=True))


_mblk = lambda: pl.BlockSpec((ROWB, C), lambda i: (i, 0))

_mlp = pl.pallas_call(
    _mlp_body,
    grid=(NP // ROWB,),
    in_specs=[
        pl.BlockSpec((ROWB, 256), lambda i: (i, 0)),
        pl.BlockSpec((256, 512), lambda i: (0, 0)),
        pl.BlockSpec((512, 256), lambda i: (0, 0)),
        pl.BlockSpec((256, C), lambda i: (0, 0)),
    ],
    out_specs=_mblk(),
    out_shape=jax.ShapeDtypeStruct((NP, C), jnp.float32),
)

_prep = pl.pallas_call(
    _prep_body,
    grid=(NP // ROWB,),
    in_specs=[_mblk(), pl.BlockSpec((NCORES, ROWB, C), lambda i: (0, i, 0))],
    out_specs=[_mblk(), _mblk(), _mblk(), _mblk()],
    out_shape=[jax.ShapeDtypeStruct((NP, C), jnp.float32)] * 4,
)

_fblk = lambda: pl.BlockSpec((FROWB, C), lambda i: (i, 0))

_final = pl.pallas_call(
    _final_body,
    grid=(N // FROWB,),
    in_specs=[pl.BlockSpec((NCORES, FROWB, C), lambda i: (0, i, 0)),
              _fblk(), _fblk(), _fblk(), _fblk()],
    out_specs=_fblk(),
    out_shape=jax.ShapeDtypeStruct((N, C), jnp.float32),
)


def kernel(adj_dense, attr_matrix, test, epochs, edge_index, W0, W1, W2):
    src = edge_index[0].astype(jnp.int32)
    dst = edge_index[1].astype(jnp.int32)
    npad = EPAD - E
    src_t = jnp.concatenate(
        [src, jnp.full((npad,), PAD_ROW, jnp.int32)]).reshape(NTILES, NCHUNK, CH)
    dst_t = jnp.concatenate(
        [dst, jnp.zeros((npad,), jnp.int32)]).reshape(NTILES, NCHUNK, CH)
    zeros_tile = jnp.zeros((RPT, C), jnp.float32)
    ones_y = jnp.ones((NP, C), jnp.float32)

    pdeg = _sc_aggregate(ones_y, dst_t, src_t, zeros_tile)
    logits = _mlp(attr_matrix, W0, W1, W2)
    y, m, c_w, sq_w = _prep(logits, pdeg)
    for k in range(10):
        p = _sc_aggregate(y, dst_t, src_t, zeros_tile)
        if k < 9:
            y = c_w * (p[0] + p[1] + y) + m
        else:
            out = _final(p, y, c_w, m, sq_w)
    return out
